# baseline (speedup 1.0000x reference)
import functools

import jax
import jax.numpy as jnp
from jax import lax
from jax.experimental import pallas as pl
from jax.experimental.pallas import tpu as pltpu
from jax.experimental.pallas import tpu_sc as plsc

N_FEATURE = 256
DEPTH = 8
N_INTERNAL = 2**DEPTH - 1
N_LEAF = 2**DEPTH
N_TREE = 100
N_BATCH = 4096
LANES = 16
NUM_WORKERS = 32
ROWS_PER_W = N_BATCH // NUM_WORKERS
REG_DEPTH = 6
N_DEEP = N_INTERNAL - 63


def _tree_kernel_body(xt_hbm, rf_hbm, rt_hbm, f67_hbm, t67_hbm, val_hbm,
                      out_hbm, x_v, rf_v, rt_v, f67_v, t67_v, val_v, out_v,
                      sem):
    c = lax.axis_index("c")
    s = lax.axis_index("s")
    wid = s * 2 + c
    base = wid * ROWS_PER_W

    cp = [pltpu.async_copy(xt_hbm.at[wid], x_v, sem),
          pltpu.async_copy(rf_hbm, rf_v, sem),
          pltpu.async_copy(rt_hbm, rt_v, sem),
          pltpu.async_copy(f67_hbm, f67_v, sem),
          pltpu.async_copy(t67_hbm, t67_v, sem),
          pltpu.async_copy(val_hbm, val_v, sem)]
    for c_ in cp:
        c_.wait()

    lane = lax.iota(jnp.int32, LANES)
    n_groups = ROWS_PER_W // LANES

    @plsc.parallel_loop(0, N_TREE, unroll=4, carry=tuple(
        jnp.zeros((LANES,), jnp.float32) for _ in range(n_groups)))
    def accs(t, accs):
        rf = [rf_v[t, pl.ds(k * LANES, LANES)] for k in range(4)]
        rt = [rt_v[t, pl.ds(k * LANES, LANES)] for k in range(4)]
        f6 = f67_v.at[pl.ds(t * N_DEEP, 64)]
        f7 = f67_v.at[pl.ds(t * N_DEEP + 64, 128)]
        t6 = t67_v.at[pl.ds(t * N_DEEP, 64)]
        t7 = t67_v.at[pl.ds(t * N_DEEP + 64, 128)]
        vw = val_v.at[pl.ds(t * N_LEAF, N_LEAF)]
        ls = [jnp.zeros((LANES,), jnp.int32)] * n_groups
        for d in range(DEPTH):
            if d < 4:
                idxs = [l + (2**d - 1) for l in ls]
                fs = [jnp.take_along_axis(rf[0], i, axis=0) for i in idxs]
                ths = [jnp.take_along_axis(rt[0], i, axis=0) for i in idxs]
            elif d == 4:
                fs = [jnp.take_along_axis(rf[1], l, axis=0) for l in ls]
                ths = [jnp.take_along_axis(rt[1], l, axis=0) for l in ls]
            elif d == 5:
                los = [l & (LANES - 1) for l in ls]
                his = [l >= LANES for l in ls]
                fs = [jnp.where(hi, jnp.take_along_axis(rf[3], lo, axis=0),
                                jnp.take_along_axis(rf[2], lo, axis=0))
                      for lo, hi in zip(los, his)]
                ths = [jnp.where(hi, jnp.take_along_axis(rt[3], lo, axis=0),
                                 jnp.take_along_axis(rt[2], lo, axis=0))
                      for lo, hi in zip(los, his)]
            elif d == 6:
                fs = [plsc.load_gather(f6, [l]) for l in ls]
                ths = [plsc.load_gather(t6, [l]) for l in ls]
            else:
                fs = [plsc.load_gather(f7, [l]) for l in ls]
                ths = [plsc.load_gather(t7, [l]) for l in ls]
            xvs = [plsc.load_gather(x_v, [fs[bg] + (lane + bg * LANES)])
                   for bg in range(n_groups)]
            ls = [2 * ls[bg] + (xvs[bg] > ths[bg]).astype(jnp.int32)
                  for bg in range(n_groups)]
        return tuple(accs[bg] + plsc.load_gather(vw, [ls[bg]])
                     for bg in range(n_groups))

    for bg in range(n_groups):
        out_v[pl.ds(bg * LANES, LANES)] = 1.0 / (1.0 + jnp.exp(-accs[bg]))

    pltpu.sync_copy(out_v, out_hbm.at[pl.ds(base, ROWS_PER_W)])


@functools.partial(jax.jit, static_argnames=())
def _run_sc(xt, rf, rt, f67, t67, val):
    mesh = plsc.VectorSubcoreMesh(core_axis_name="c", subcore_axis_name="s")
    call = pl.kernel(
        _tree_kernel_body,
        out_type=jax.ShapeDtypeStruct((N_BATCH,), jnp.float32),
        mesh=mesh,
        scratch_types=[
            pltpu.VMEM((N_FEATURE * ROWS_PER_W,), jnp.float32),
            pltpu.VMEM((N_TREE, 64), jnp.int32),
            pltpu.VMEM((N_TREE, 64), jnp.float32),
            pltpu.VMEM((N_TREE * N_DEEP,), jnp.int32),
            pltpu.VMEM((N_TREE * N_DEEP,), jnp.float32),
            pltpu.VMEM((N_TREE * N_LEAF,), jnp.float32),
            pltpu.VMEM((ROWS_PER_W,), jnp.float32),
            pltpu.SemaphoreType.DMA,
        ],
        compiler_params=pltpu.CompilerParams(use_tc_tiling_on_sc=False,
                                             needs_layout_passes=False),
    )
    return call(xt, rf, rt, f67, t67, val)


def kernel(x, feature, threshold, children_left, children_right, value):
    del children_left, children_right
    n_batch, _ = x.shape
    zpad_i = jnp.zeros((N_TREE, 1), jnp.int32)
    zpad_f = jnp.zeros((N_TREE, 1), jnp.float32)
    feat = feature.astype(jnp.int32) * ROWS_PER_W
    rf = jnp.concatenate(
        [feat[:, 0:15], zpad_i, feat[:, 15:31], feat[:, 31:63]], axis=1)
    rt = jnp.concatenate(
        [threshold[:, 0:15], zpad_f, threshold[:, 15:31],
         threshold[:, 31:63]], axis=1)
    f67 = feat[:, 63:N_INTERNAL].reshape(-1)
    t67 = threshold[:, 63:N_INTERNAL].reshape(-1)
    val = value[:, N_INTERNAL:, 0].astype(jnp.float32).reshape(-1)
    xw = jnp.transpose(x.reshape(NUM_WORKERS, ROWS_PER_W, N_FEATURE),
                       (0, 2, 1)).reshape(NUM_WORKERS, -1)
    out = _run_sc(xw, rf, rt, f67, t67, val)
    return out.reshape(n_batch, 1)


# speedup vs baseline: 1.0970x; 1.0970x over previous
import functools

import jax
import jax.numpy as jnp
from jax import lax
from jax.experimental import pallas as pl
from jax.experimental.pallas import tpu as pltpu
from jax.experimental.pallas import tpu_sc as plsc

N_FEATURE = 256
DEPTH = 8
N_INTERNAL = 2**DEPTH - 1
N_LEAF = 2**DEPTH
N_TREE = 100
N_BATCH = 4096
LANES = 16
NUM_WORKERS = 32
ROWS_PER_W = N_BATCH // NUM_WORKERS


def _tree_kernel_body(xw_hbm, feat_hbm, thr_hbm, val_hbm, out_hbm,
                      x_v, feat_v, thr_v, val_v, out_v, sem):
    c = lax.axis_index("c")
    s = lax.axis_index("s")
    wid = s * 2 + c
    base = wid * ROWS_PER_W

    cp = [pltpu.async_copy(xw_hbm.at[wid], x_v, sem),
          pltpu.async_copy(feat_hbm.at[:, pl.ds(0, N_LEAF)], feat_v, sem),
          pltpu.async_copy(thr_hbm.at[:, pl.ds(0, N_LEAF)], thr_v, sem),
          pltpu.async_copy(val_hbm.at[:, pl.ds(248, 264)], val_v, sem)]
    for c_ in cp:
        c_.wait()

    lane = lax.iota(jnp.int32, LANES)
    n_groups = ROWS_PER_W // LANES

    @plsc.parallel_loop(0, N_TREE, unroll=2, carry=tuple(
        jnp.zeros((LANES,), jnp.float32) for _ in range(n_groups)))
    def accs(t, accs):
        rf = [feat_v[t, pl.ds(k * LANES, LANES)] for k in range(4)]
        rt = [thr_v[t, pl.ds(k * LANES, LANES)] for k in range(4)]
        f6 = feat_v.at[t, pl.ds(56, 72)]
        f7 = feat_v.at[t, pl.ds(120, 136)]
        t6 = thr_v.at[t, pl.ds(56, 72)]
        t7 = thr_v.at[t, pl.ds(120, 136)]
        vw = val_v.at[t]
        ls = [jnp.zeros((LANES,), jnp.int32)] * n_groups
        for d in range(DEPTH):
            if d < 4:
                idxs = [l + (2**d - 1) for l in ls]
                fs = [jnp.take_along_axis(rf[0], i, axis=0) for i in idxs]
                ths = [jnp.take_along_axis(rt[0], i, axis=0) for i in idxs]
            elif d == 4:
                los = [(l + 15) & (LANES - 1) for l in ls]
                his = [l >= 1 for l in ls]
                fs = [jnp.where(hi, jnp.take_along_axis(rf[1], lo, axis=0),
                                jnp.take_along_axis(rf[0], lo, axis=0))
                      for lo, hi in zip(los, his)]
                ths = [jnp.where(hi, jnp.take_along_axis(rt[1], lo, axis=0),
                                 jnp.take_along_axis(rt[0], lo, axis=0))
                       for lo, hi in zip(los, his)]
            elif d == 5:
                los = [(l + 31) & (LANES - 1) for l in ls]
                mids = [l >= 1 for l in ls]
                his = [l >= 17 for l in ls]
                fs = [jnp.where(hi, jnp.take_along_axis(rf[3], lo, axis=0),
                                jnp.where(mid,
                                          jnp.take_along_axis(rf[2], lo,
                                                              axis=0),
                                          jnp.take_along_axis(rf[1], lo,
                                                              axis=0)))
                      for lo, mid, hi in zip(los, mids, his)]
                ths = [jnp.where(hi, jnp.take_along_axis(rt[3], lo, axis=0),
                                 jnp.where(mid,
                                           jnp.take_along_axis(rt[2], lo,
                                                               axis=0),
                                           jnp.take_along_axis(rt[1], lo,
                                                               axis=0)))
                       for lo, mid, hi in zip(los, mids, his)]
            elif d == 6:
                i6 = [l + 7 for l in ls]
                fs = [plsc.load_gather(f6, [i]) for i in i6]
                ths = [plsc.load_gather(t6, [i]) for i in i6]
            else:
                i7 = [l + 7 for l in ls]
                fs = [plsc.load_gather(f7, [i]) for i in i7]
                ths = [plsc.load_gather(t7, [i]) for i in i7]
            xvs = [plsc.load_gather(x_v, [fs[bg] + (lane + bg * LANES)])
                   for bg in range(n_groups)]
            ls = [2 * ls[bg] + (xvs[bg] > ths[bg]).astype(jnp.int32)
                  for bg in range(n_groups)]
        return tuple(accs[bg] + plsc.load_gather(vw, [ls[bg] + 7])
                     for bg in range(n_groups))

    for bg in range(n_groups):
        out_v[pl.ds(bg * LANES, LANES)] = 1.0 / (1.0 + jnp.exp(-accs[bg]))

    pltpu.sync_copy(out_v, out_hbm.at[pl.ds(base, ROWS_PER_W)])


@functools.partial(jax.jit, static_argnames=())
def _run_sc(xw, fpre, thr, val2d):
    mesh = plsc.VectorSubcoreMesh(core_axis_name="c", subcore_axis_name="s")
    call = pl.kernel(
        _tree_kernel_body,
        out_type=jax.ShapeDtypeStruct((N_BATCH,), jnp.float32),
        mesh=mesh,
        scratch_types=[
            pltpu.VMEM((N_FEATURE * ROWS_PER_W,), jnp.float32),
            pltpu.VMEM((N_TREE, N_LEAF), jnp.int32),
            pltpu.VMEM((N_TREE, N_LEAF), jnp.float32),
            pltpu.VMEM((N_TREE, 264), jnp.float32),
            pltpu.VMEM((ROWS_PER_W,), jnp.float32),
            pltpu.SemaphoreType.DMA,
        ],
        compiler_params=pltpu.CompilerParams(use_tc_tiling_on_sc=False,
                                             needs_layout_passes=False),
    )
    return call(xw, fpre, thr, val2d)


def kernel(x, feature, threshold, children_left, children_right, value):
    del children_left, children_right
    n_batch, _ = x.shape
    fpre = feature.astype(jnp.int32) * ROWS_PER_W
    xw = jnp.transpose(x.reshape(NUM_WORKERS, ROWS_PER_W, N_FEATURE),
                       (0, 2, 1)).reshape(NUM_WORKERS, -1)
    val2d = jnp.pad(value[:, :, 0], ((0, 0), (0, 1)))
    out = _run_sc(xw, fpre, threshold, val2d)
    return out.reshape(n_batch, 1)


# speedup vs baseline: 1.1705x; 1.0670x over previous
import functools

import jax
import jax.numpy as jnp
import numpy as np
from jax import lax
from jax.experimental import pallas as pl
from jax.experimental.pallas import tpu as pltpu
from jax.experimental.pallas import tpu_sc as plsc

N_FEATURE = 256
DEPTH = 8
N_INTERNAL = 2**DEPTH - 1
N_LEAF = 2**DEPTH
N_TREE = 100
N_BATCH = 4096
LANES = 16
NUM_WORKERS = 32
ROWS_PER_W = N_BATCH // NUM_WORKERS

_IDXF = np.concatenate([np.arange(0, 15), [0], np.arange(15, 31),
                        np.arange(31, 63), np.arange(63, 255)]).astype(
                            np.int32)


def _tree_kernel_body(xw_hbm, fi_hbm, ft_hbm, val_hbm, out_hbm,
                      x_v, fi_v, ft_v, val_v, out_v, sem):
    c = lax.axis_index("c")
    s = lax.axis_index("s")
    wid = s * 2 + c
    base = wid * ROWS_PER_W

    cp = [pltpu.async_copy(xw_hbm.at[wid], x_v, sem),
          pltpu.async_copy(fi_hbm, fi_v, sem),
          pltpu.async_copy(ft_hbm, ft_v, sem),
          pltpu.async_copy(val_hbm, val_v, sem)]
    for c_ in cp:
        c_.wait()

    lane = lax.iota(jnp.int32, LANES)
    n_groups = ROWS_PER_W // LANES

    @plsc.parallel_loop(0, N_TREE, unroll=2, carry=tuple(
        jnp.zeros((LANES,), jnp.float32) for _ in range(n_groups)))
    def accs(t, accs):
        rf = [fi_v[t, pl.ds(k * LANES, LANES)] for k in range(4)]
        rt = [ft_v[t, pl.ds(k * LANES, LANES)] for k in range(4)]
        f6 = fi_v.at[t, pl.ds(64, 64)]
        f7 = fi_v.at[t, pl.ds(128, 128)]
        t6 = ft_v.at[t, pl.ds(64, 64)]
        t7 = ft_v.at[t, pl.ds(128, 128)]
        vw = val_v.at[t]
        ls = [jnp.zeros((LANES,), jnp.int32)] * n_groups
        for d in range(DEPTH):
            if d < 4:
                idxs = [l + (2**d - 1) for l in ls]
                fs = [jnp.take_along_axis(rf[0], i, axis=0) for i in idxs]
                ths = [jnp.take_along_axis(rt[0], i, axis=0) for i in idxs]
            elif d == 4:
                fs = [jnp.take_along_axis(rf[1], l, axis=0) for l in ls]
                ths = [jnp.take_along_axis(rt[1], l, axis=0) for l in ls]
            elif d == 5:
                los = [l & (LANES - 1) for l in ls]
                his = [l >= LANES for l in ls]
                fs = [jnp.where(hi, jnp.take_along_axis(rf[3], lo, axis=0),
                                jnp.take_along_axis(rf[2], lo, axis=0))
                      for lo, hi in zip(los, his)]
                ths = [jnp.where(hi, jnp.take_along_axis(rt[3], lo, axis=0),
                                 jnp.take_along_axis(rt[2], lo, axis=0))
                       for lo, hi in zip(los, his)]
            elif d == 6:
                fs = [plsc.load_gather(f6, [l]) for l in ls]
                ths = [plsc.load_gather(t6, [l]) for l in ls]
            else:
                fs = [plsc.load_gather(f7, [l]) for l in ls]
                ths = [plsc.load_gather(t7, [l]) for l in ls]
            xvs = [plsc.load_gather(x_v, [fs[bg] + (lane + bg * LANES)])
                   for bg in range(n_groups)]
            ls = [2 * ls[bg] + (xvs[bg] > ths[bg]).astype(jnp.int32)
                  for bg in range(n_groups)]
        return tuple(accs[bg] + plsc.load_gather(vw, [ls[bg]])
                     for bg in range(n_groups))

    for bg in range(n_groups):
        out_v[pl.ds(bg * LANES, LANES)] = 1.0 / (1.0 + jnp.exp(-accs[bg]))

    pltpu.sync_copy(out_v, out_hbm.at[pl.ds(base, ROWS_PER_W)])


@functools.partial(jax.jit, static_argnames=())
def _run_sc(xw, fi, ft, val):
    mesh = plsc.VectorSubcoreMesh(core_axis_name="c", subcore_axis_name="s")
    call = pl.kernel(
        _tree_kernel_body,
        out_type=jax.ShapeDtypeStruct((N_BATCH,), jnp.float32),
        mesh=mesh,
        scratch_types=[
            pltpu.VMEM((N_FEATURE * ROWS_PER_W,), jnp.float32),
            pltpu.VMEM((N_TREE, N_LEAF), jnp.int32),
            pltpu.VMEM((N_TREE, N_LEAF), jnp.float32),
            pltpu.VMEM((N_TREE, N_LEAF), jnp.float32),
            pltpu.VMEM((ROWS_PER_W,), jnp.float32),
            pltpu.SemaphoreType.DMA,
        ],
        compiler_params=pltpu.CompilerParams(use_tc_tiling_on_sc=False,
                                             needs_layout_passes=False),
    )
    return call(xw, fi, ft, val)


def kernel(x, feature, threshold, children_left, children_right, value):
    del children_left, children_right
    n_batch, _ = x.shape
    idx = jnp.asarray(_IDXF)
    fi = jnp.take(feature.astype(jnp.int32) * ROWS_PER_W, idx, axis=1)
    ft = jnp.take(threshold, idx, axis=1)
    val = value[:, N_INTERNAL:, 0]
    xw = jnp.transpose(x.reshape(NUM_WORKERS, ROWS_PER_W, N_FEATURE),
                       (0, 2, 1)).reshape(NUM_WORKERS, -1)
    out = _run_sc(xw, fi, ft, val)
    return out.reshape(n_batch, 1)
